# Initial kernel scaffold; baseline (speedup 1.0000x reference)
import jax
import jax.numpy as jnp
from jax.experimental import pallas as pl


def kernel(x, edge_index, W1, b1, W2, b2):
    raise NotImplementedError("write your pallas kernel here")


# speedup vs baseline: 14.2215x; 14.2215x over previous
import functools

import jax
import jax.numpy as jnp
from jax import lax
from jax.experimental import pallas as pl
from jax.experimental.pallas import tpu as pltpu
from jax.experimental.pallas import tpu_sc as plsc

NCORE = 2
NSUB = 16
EB = 125
FC = 128
RB = 1024
NPAD = 10240


IB = 8


def _make_deg(E):
    ROWS = E // EB
    RPT = ROWS // NSUB
    NG = RPT // IB
    RN = NPAD // NSUB
    mesh = plsc.VectorSubcoreMesh(core_axis_name="c", subcore_axis_name="s",
                                  num_cores=NCORE, num_subcores=NSUB)

    @functools.partial(
        pl.kernel,
        out_type=jax.ShapeDtypeStruct((NCORE, NPAD), jnp.float32),
        mesh=mesh,
        scratch_types=[
            pltpu.VMEM((IB, EB), jnp.int32),
            pltpu.VMEM((EB,), jnp.float32),
            pltpu.VMEM((RN,), jnp.float32),
            pltpu.VMEM_SHARED((NPAD,), jnp.float32),
        ],
    )
    def deg_kernel(dst_hbm, ones_hbm, zeros_hbm, out_hbm, idxb, ones_v, zv, acc):
        cid = lax.axis_index("c")
        sid = lax.axis_index("s")
        pltpu.sync_copy(ones_hbm, ones_v)
        pltpu.sync_copy(zeros_hbm.at[pl.ds(sid * RN, RN)], zv)
        pltpu.sync_copy(zv, acc.at[pl.ds(sid * RN, RN)])
        plsc.subcore_barrier()

        def group(g, carry):
            base = pl.multiple_of(sid * RPT + g * IB, IB)
            pltpu.sync_copy(dst_hbm.at[pl.ds(base, IB)], idxb)
            for j in range(IB):
                pltpu.sync_copy(ones_v, acc.at[idxb.at[j]], add=True)
            return carry

        @pl.when(cid == 0)
        def _():
            lax.fori_loop(0, NG // 2, group, 0)

        @pl.when(cid == 1)
        def _():
            lax.fori_loop(NG // 2, NG, group, 0)

        plsc.subcore_barrier()
        pltpu.sync_copy(acc.at[pl.ds(sid * RN, RN)],
                        out_hbm.at[cid, pl.ds(sid * RN, RN)])

    return deg_kernel


def _make_gather_scatter(E, nchunks):
    ROWS = E // EB
    RPT = ROWS // NSUB
    RN = NPAD // NSUB
    CPC = nchunks // NCORE
    mesh = plsc.VectorSubcoreMesh(core_axis_name="c", subcore_axis_name="s",
                                  num_cores=NCORE, num_subcores=NSUB)

    @functools.partial(
        pl.kernel,
        out_type=jax.ShapeDtypeStruct((nchunks, NPAD, FC), jnp.float32),
        mesh=mesh,
        scratch_types=[
            pltpu.VMEM((IB, EB), jnp.int32),
            pltpu.VMEM((IB, EB), jnp.int32),
            pltpu.VMEM((EB, FC), jnp.float32),
            pltpu.VMEM((EB, FC), jnp.float32),
            pltpu.VMEM_SHARED((NPAD, FC), jnp.float32),
            pltpu.SemaphoreType.DMA,
            pltpu.SemaphoreType.DMA,
        ],
    )
    def gs_kernel(g_hbm, src_hbm, dst_hbm, out_hbm,
                  sbuf, dbuf, rows0, rows1, acc, sem0, sem1):
        cid = lax.axis_index("c")
        sid = lax.axis_index("s")

        def run_chunk(chunk):
            gc = g_hbm.at[chunk]
            pltpu.sync_copy(g_hbm.at[chunk, pl.ds(sid * RN, RN)],
                            acc.at[pl.ds(sid * RN, RN)])
            plsc.subcore_barrier()

            def group(g, carry):
                base = pl.multiple_of(sid * RPT + g * IB, IB)
                pltpu.sync_copy(src_hbm.at[pl.ds(base, IB)], sbuf)
                pltpu.sync_copy(dst_hbm.at[pl.ds(base, IB)], dbuf)
                bufs = ((rows0, sem0), (rows1, sem1))
                pltpu.async_copy(gc.at[sbuf.at[0]], rows0, sem0)
                for j in range(IB):
                    r_cur, s_cur = bufs[j % 2]
                    r_nxt, s_nxt = bufs[(j + 1) % 2]
                    if j + 1 < IB:
                        pltpu.async_copy(gc.at[sbuf.at[j + 1]], r_nxt, s_nxt)
                    pltpu.make_async_copy(gc.at[sbuf.at[j]], r_cur, s_cur).wait()
                    pltpu.sync_copy(r_cur, acc.at[dbuf.at[j]], add=True)
                return carry

            lax.fori_loop(0, RPT // IB, group, 0)
            plsc.subcore_barrier()
            pltpu.sync_copy(acc.at[pl.ds(sid * RN, RN)],
                            out_hbm.at[chunk, pl.ds(sid * RN, RN)])
            plsc.subcore_barrier()

        @pl.when(cid == 0)
        def _():
            for chunk in range(CPC):
                run_chunk(chunk)

        @pl.when(cid == 1)
        def _():
            for chunk in range(CPC, 2 * CPC):
                run_chunk(chunk)

    return gs_kernel


def _dinv_block(p_ref):
    deg = p_ref[0, :] + p_ref[1, :] + 1.0
    return lax.rsqrt(deg)[:, None]


def _mm1(x, W1, degp):
    N, K = x.shape
    nc1 = W1.shape[1] // FC

    def body(x_ref, w_ref, p_ref, o_ref):
        dinv = _dinv_block(p_ref)
        o_ref[0] = jnp.dot(x_ref[...], w_ref[...],
                           preferred_element_type=jnp.float32) * dinv

    return pl.pallas_call(
        body,
        grid=(N // RB, nc1),
        in_specs=[
            pl.BlockSpec((RB, K), lambda i, c: (i, 0)),
            pl.BlockSpec((K, FC), lambda i, c: (0, c)),
            pl.BlockSpec((2, RB), lambda i, c: (0, i)),
        ],
        out_specs=pl.BlockSpec((1, RB, FC), lambda i, c: (c, i, 0)),
        out_shape=jax.ShapeDtypeStruct((nc1, N, FC), jnp.float32),
    )(x, W1, degp)


def _mm2(T1, degp, b1, W2):
    nc1, N, _ = T1.shape
    F1 = nc1 * FC
    zc = W2.shape[1] // FC

    def body(t_ref, p_ref, b_ref, w_ref, o_ref):
        dinv = _dinv_block(p_ref)
        acc = jnp.zeros((RB, FC), jnp.float32)
        for ci in range(nc1):
            h = jnp.maximum(t_ref[ci] * dinv + b_ref[0, ci * FC:(ci + 1) * FC][None, :], 0.0)
            acc = acc + jnp.dot(h, w_ref[ci * FC:(ci + 1) * FC, :],
                                preferred_element_type=jnp.float32)
        o_ref[0] = acc * dinv

    return pl.pallas_call(
        body,
        grid=(N // RB, zc),
        in_specs=[
            pl.BlockSpec((nc1, RB, FC), lambda i, c: (0, i, 0)),
            pl.BlockSpec((2, RB), lambda i, c: (0, i)),
            pl.BlockSpec((1, F1), lambda i, c: (0, 0)),
            pl.BlockSpec((F1, FC), lambda i, c: (0, c)),
        ],
        out_specs=pl.BlockSpec((1, RB, FC), lambda i, c: (c, i, 0)),
        out_shape=jax.ShapeDtypeStruct((zc, N, FC), jnp.float32),
    )(T1, degp, b1.reshape(1, F1), W2)


def _combine(T2, degp, b2):
    zc, N, _ = T2.shape

    def body(t_ref, p_ref, b_ref, o_ref):
        dinv = _dinv_block(p_ref)
        o_ref[...] = t_ref[0] * dinv + b_ref[0][None, :]

    return pl.pallas_call(
        body,
        grid=(N // RB, zc),
        in_specs=[
            pl.BlockSpec((1, RB, FC), lambda i, c: (c, i, 0)),
            pl.BlockSpec((2, RB), lambda i, c: (0, i)),
            pl.BlockSpec((1, FC), lambda i, c: (0, c)),
        ],
        out_specs=pl.BlockSpec((RB, FC), lambda i, c: (i, c)),
        out_shape=jax.ShapeDtypeStruct((N, zc * FC), jnp.float32),
    )(T2, degp, b2.reshape(1, -1))


def kernel(x, edge_index, W1, b1, W2, b2):
    N = x.shape[0]
    E = edge_index.shape[1]
    xp = jnp.pad(x, ((0, NPAD - N), (0, 0)))
    src2d = edge_index[0].reshape(E // EB, EB)
    dst2d = edge_index[1].reshape(E // EB, EB)
    ones = jnp.ones((EB,), jnp.float32)
    zeros = jnp.zeros((NPAD,), jnp.float32)

    degp = _make_deg(E)(dst2d, ones, zeros)
    G1 = _mm1(xp, W1, degp)
    T1 = _make_gather_scatter(E, W1.shape[1] // FC)(G1, src2d, dst2d)
    G2 = _mm2(T1, degp, b1, W2)
    T2 = _make_gather_scatter(E, W2.shape[1] // FC)(G2, src2d, dst2d)
    return _combine(T2, degp, b2)[:N]


# speedup vs baseline: 14.2490x; 1.0019x over previous
import functools

import jax
import jax.numpy as jnp
from jax import lax
from jax.experimental import pallas as pl
from jax.experimental.pallas import tpu as pltpu
from jax.experimental.pallas import tpu_sc as plsc

NCORE = 2
NSUB = 16
EB = 125
FC = 128
RB = 1024
NPAD = 10240


IB = 8


def _make_deg(E):
    ROWS = E // EB
    RPT = ROWS // NSUB
    NG = RPT // IB
    RN = NPAD // NSUB
    mesh = plsc.VectorSubcoreMesh(core_axis_name="c", subcore_axis_name="s",
                                  num_cores=NCORE, num_subcores=NSUB)

    @functools.partial(
        pl.kernel,
        out_type=jax.ShapeDtypeStruct((NCORE, NPAD), jnp.float32),
        mesh=mesh,
        scratch_types=[
            pltpu.VMEM((IB, EB), jnp.int32),
            pltpu.VMEM((EB,), jnp.float32),
            pltpu.VMEM((RN,), jnp.float32),
            pltpu.VMEM_SHARED((NPAD,), jnp.float32),
        ],
    )
    def deg_kernel(dst_hbm, ones_hbm, zeros_hbm, out_hbm, idxb, ones_v, zv, acc):
        cid = lax.axis_index("c")
        sid = lax.axis_index("s")
        pltpu.sync_copy(ones_hbm, ones_v)
        pltpu.sync_copy(zeros_hbm.at[pl.ds(sid * RN, RN)], zv)
        pltpu.sync_copy(zv, acc.at[pl.ds(sid * RN, RN)])
        plsc.subcore_barrier()

        def group(g, carry):
            base = pl.multiple_of(sid * RPT + g * IB, IB)
            pltpu.sync_copy(dst_hbm.at[pl.ds(base, IB)], idxb)
            for j in range(IB):
                pltpu.sync_copy(ones_v, acc.at[idxb.at[j]], add=True)
            return carry

        @pl.when(cid == 0)
        def _():
            lax.fori_loop(0, NG // 2, group, 0)

        @pl.when(cid == 1)
        def _():
            lax.fori_loop(NG // 2, NG, group, 0)

        plsc.subcore_barrier()
        pltpu.sync_copy(acc.at[pl.ds(sid * RN, RN)],
                        out_hbm.at[cid, pl.ds(sid * RN, RN)])

    return deg_kernel


def _make_gather_scatter(E, nchunks):
    ROWS = E // EB
    RPT = ROWS // NSUB
    RN = NPAD // NSUB
    CPC = nchunks // NCORE
    mesh = plsc.VectorSubcoreMesh(core_axis_name="c", subcore_axis_name="s",
                                  num_cores=NCORE, num_subcores=NSUB)

    @functools.partial(
        pl.kernel,
        out_type=jax.ShapeDtypeStruct((nchunks, NPAD, FC), jnp.float32),
        mesh=mesh,
        scratch_types=[
            pltpu.VMEM((IB, EB), jnp.int32),
            pltpu.VMEM((IB, EB), jnp.int32),
            pltpu.VMEM((EB, FC), jnp.float32),
            pltpu.VMEM((EB, FC), jnp.float32),
            pltpu.VMEM_SHARED((NPAD, FC), jnp.float32),
            pltpu.SemaphoreType.DMA,
            pltpu.SemaphoreType.DMA,
            pltpu.SemaphoreType.DMA,
            pltpu.SemaphoreType.DMA,
        ],
    )
    def gs_kernel(g_hbm, src_hbm, dst_hbm, out_hbm,
                  sbuf, dbuf, rows0, rows1, acc, sem0, sem1, ssem0, ssem1):
        cid = lax.axis_index("c")
        sid = lax.axis_index("s")

        def run_chunk(chunk):
            gc = g_hbm.at[chunk]
            pltpu.sync_copy(g_hbm.at[chunk, pl.ds(sid * RN, RN)],
                            acc.at[pl.ds(sid * RN, RN)])
            plsc.subcore_barrier()

            def group(g, carry):
                base = pl.multiple_of(sid * RPT + g * IB, IB)
                pltpu.sync_copy(src_hbm.at[pl.ds(base, IB)], sbuf)
                pltpu.sync_copy(dst_hbm.at[pl.ds(base, IB)], dbuf)
                bufs = ((rows0, sem0, ssem0), (rows1, sem1, ssem1))
                scats = {}
                pltpu.async_copy(gc.at[sbuf.at[0]], rows0, sem0)
                for j in range(IB):
                    r_cur, s_cur, ss_cur = bufs[j % 2]
                    r_nxt, s_nxt, _ = bufs[(j + 1) % 2]
                    if j + 1 < IB:
                        if j >= 1:
                            scats[j - 1].wait()
                        pltpu.async_copy(gc.at[sbuf.at[j + 1]], r_nxt, s_nxt)
                    pltpu.make_async_copy(gc.at[sbuf.at[j]], r_cur, s_cur).wait()
                    scats[j] = pltpu.async_copy(r_cur, acc.at[dbuf.at[j]],
                                                ss_cur, add=True)
                scats[IB - 2].wait()
                scats[IB - 1].wait()
                return carry

            lax.fori_loop(0, RPT // IB, group, 0)
            plsc.subcore_barrier()
            pltpu.sync_copy(acc.at[pl.ds(sid * RN, RN)],
                            out_hbm.at[chunk, pl.ds(sid * RN, RN)])
            plsc.subcore_barrier()

        @pl.when(cid == 0)
        def _():
            for chunk in range(CPC):
                run_chunk(chunk)

        @pl.when(cid == 1)
        def _():
            for chunk in range(CPC, 2 * CPC):
                run_chunk(chunk)

    return gs_kernel


def _dinv_block(p_ref):
    deg = p_ref[0, :] + p_ref[1, :] + 1.0
    return lax.rsqrt(deg)[:, None]


def _mm1(x, W1, degp):
    N, K = x.shape
    nc1 = W1.shape[1] // FC

    def body(x_ref, w_ref, p_ref, o_ref):
        dinv = _dinv_block(p_ref)
        o_ref[0] = jnp.dot(x_ref[...], w_ref[...],
                           preferred_element_type=jnp.float32) * dinv

    return pl.pallas_call(
        body,
        grid=(N // RB, nc1),
        in_specs=[
            pl.BlockSpec((RB, K), lambda i, c: (i, 0)),
            pl.BlockSpec((K, FC), lambda i, c: (0, c)),
            pl.BlockSpec((2, RB), lambda i, c: (0, i)),
        ],
        out_specs=pl.BlockSpec((1, RB, FC), lambda i, c: (c, i, 0)),
        out_shape=jax.ShapeDtypeStruct((nc1, N, FC), jnp.float32),
    )(x, W1, degp)


def _mm2(T1, degp, b1, W2):
    nc1, N, _ = T1.shape
    F1 = nc1 * FC
    zc = W2.shape[1] // FC

    def body(t_ref, p_ref, b_ref, w_ref, o_ref):
        dinv = _dinv_block(p_ref)
        acc = jnp.zeros((RB, FC), jnp.float32)
        for ci in range(nc1):
            h = jnp.maximum(t_ref[ci] * dinv + b_ref[0, ci * FC:(ci + 1) * FC][None, :], 0.0)
            acc = acc + jnp.dot(h, w_ref[ci * FC:(ci + 1) * FC, :],
                                preferred_element_type=jnp.float32)
        o_ref[0] = acc * dinv

    return pl.pallas_call(
        body,
        grid=(N // RB, zc),
        in_specs=[
            pl.BlockSpec((nc1, RB, FC), lambda i, c: (0, i, 0)),
            pl.BlockSpec((2, RB), lambda i, c: (0, i)),
            pl.BlockSpec((1, F1), lambda i, c: (0, 0)),
            pl.BlockSpec((F1, FC), lambda i, c: (0, c)),
        ],
        out_specs=pl.BlockSpec((1, RB, FC), lambda i, c: (c, i, 0)),
        out_shape=jax.ShapeDtypeStruct((zc, N, FC), jnp.float32),
    )(T1, degp, b1.reshape(1, F1), W2)


def _combine(T2, degp, b2):
    zc, N, _ = T2.shape

    def body(t_ref, p_ref, b_ref, o_ref):
        dinv = _dinv_block(p_ref)
        o_ref[...] = t_ref[0] * dinv + b_ref[0][None, :]

    return pl.pallas_call(
        body,
        grid=(N // RB, zc),
        in_specs=[
            pl.BlockSpec((1, RB, FC), lambda i, c: (c, i, 0)),
            pl.BlockSpec((2, RB), lambda i, c: (0, i)),
            pl.BlockSpec((1, FC), lambda i, c: (0, c)),
        ],
        out_specs=pl.BlockSpec((RB, FC), lambda i, c: (i, c)),
        out_shape=jax.ShapeDtypeStruct((N, zc * FC), jnp.float32),
    )(T2, degp, b2.reshape(1, -1))


def kernel(x, edge_index, W1, b1, W2, b2):
    N = x.shape[0]
    E = edge_index.shape[1]
    xp = jnp.pad(x, ((0, NPAD - N), (0, 0)))
    src2d = edge_index[0].reshape(E // EB, EB)
    dst2d = edge_index[1].reshape(E // EB, EB)
    ones = jnp.ones((EB,), jnp.float32)
    zeros = jnp.zeros((NPAD,), jnp.float32)

    degp = _make_deg(E)(dst2d, ones, zeros)
    G1 = _mm1(xp, W1, degp)
    T1 = _make_gather_scatter(E, W1.shape[1] // FC)(G1, src2d, dst2d)
    G2 = _mm2(T1, degp, b1, W2)
    T2 = _make_gather_scatter(E, W2.shape[1] // FC)(G2, src2d, dst2d)
    return _combine(T2, degp, b2)[:N]


# speedup vs baseline: 14.3829x; 1.0094x over previous
import functools

import jax
import jax.numpy as jnp
from jax import lax
from jax.experimental import pallas as pl
from jax.experimental.pallas import tpu as pltpu
from jax.experimental.pallas import tpu_sc as plsc

NCORE = 2
NSUB = 16
EB = 125
FC = 128
RB = 1024
NPAD = 10240


IB = 8


def _make_deg(E):
    ROWS = E // EB
    RPT = ROWS // NSUB
    NG = RPT // IB
    RN = NPAD // NSUB
    mesh = plsc.VectorSubcoreMesh(core_axis_name="c", subcore_axis_name="s",
                                  num_cores=NCORE, num_subcores=NSUB)

    @functools.partial(
        pl.kernel,
        out_type=jax.ShapeDtypeStruct((NCORE, NPAD), jnp.float32),
        mesh=mesh,
        scratch_types=[
            pltpu.VMEM((IB, EB), jnp.int32),
            pltpu.VMEM((EB,), jnp.float32),
            pltpu.VMEM((RN,), jnp.float32),
            pltpu.VMEM_SHARED((NPAD,), jnp.float32),
        ],
    )
    def deg_kernel(edge_hbm, ones_hbm, zeros_hbm, out_hbm, idxb, ones_v, zv, acc):
        cid = lax.axis_index("c")
        sid = lax.axis_index("s")
        pltpu.sync_copy(ones_hbm, ones_v)
        pltpu.sync_copy(zeros_hbm.at[pl.ds(sid * RN, RN)], zv)
        pltpu.sync_copy(zv, acc.at[pl.ds(sid * RN, RN)])
        plsc.subcore_barrier()

        def group(g, carry):
            base = pl.multiple_of(ROWS + sid * RPT + g * IB, IB)
            pltpu.sync_copy(edge_hbm.at[pl.ds(base, IB)], idxb)
            for j in range(IB):
                pltpu.sync_copy(ones_v, acc.at[idxb.at[j]], add=True)
            return carry

        @pl.when(cid == 0)
        def _():
            lax.fori_loop(0, NG // 2, group, 0)

        @pl.when(cid == 1)
        def _():
            lax.fori_loop(NG // 2, NG, group, 0)

        plsc.subcore_barrier()
        pltpu.sync_copy(acc.at[pl.ds(sid * RN, RN)],
                        out_hbm.at[cid, pl.ds(sid * RN, RN)])

    return deg_kernel


def _make_gather_scatter(E, nchunks, pair):
    ROWS = E // EB
    RPT = ROWS // NSUB
    RN = NPAD // NSUB
    mesh = plsc.VectorSubcoreMesh(core_axis_name="c", subcore_axis_name="s",
                                  num_cores=NCORE, num_subcores=NSUB)

    @functools.partial(
        pl.kernel,
        out_type=jax.ShapeDtypeStruct((NCORE, NPAD, FC), jnp.float32),
        mesh=mesh,
        scratch_types=[
            pltpu.VMEM((IB, EB), jnp.int32),
            pltpu.VMEM((IB, EB), jnp.int32),
            pltpu.VMEM((EB, FC), jnp.float32),
            pltpu.VMEM((EB, FC), jnp.float32),
            pltpu.VMEM_SHARED((NPAD, FC), jnp.float32),
            pltpu.SemaphoreType.DMA,
            pltpu.SemaphoreType.DMA,
            pltpu.SemaphoreType.DMA,
            pltpu.SemaphoreType.DMA,
        ],
    )
    def gs_kernel(g_hbm, edge_hbm, out_hbm,
                  sbuf, dbuf, rows0, rows1, acc, sem0, sem1, ssem0, ssem1):
        cid = lax.axis_index("c")
        sid = lax.axis_index("s")

        def run_chunk(chunk, slot):
            gc = g_hbm.at[chunk]
            pltpu.sync_copy(g_hbm.at[chunk, pl.ds(sid * RN, RN)],
                            acc.at[pl.ds(sid * RN, RN)])
            plsc.subcore_barrier()

            def group(g, carry):
                base = pl.multiple_of(sid * RPT + g * IB, IB)
                pltpu.sync_copy(edge_hbm.at[pl.ds(base, IB)], sbuf)
                pltpu.sync_copy(edge_hbm.at[pl.ds(base + ROWS, IB)], dbuf)
                bufs = ((rows0, sem0, ssem0), (rows1, sem1, ssem1))
                scats = {}
                pltpu.async_copy(gc.at[sbuf.at[0]], rows0, sem0)
                for j in range(IB):
                    r_cur, s_cur, ss_cur = bufs[j % 2]
                    r_nxt, s_nxt, _ = bufs[(j + 1) % 2]
                    if j + 1 < IB:
                        if j >= 1:
                            scats[j - 1].wait()
                        pltpu.async_copy(gc.at[sbuf.at[j + 1]], r_nxt, s_nxt)
                    pltpu.make_async_copy(gc.at[sbuf.at[j]], r_cur, s_cur).wait()
                    scats[j] = pltpu.async_copy(r_cur, acc.at[dbuf.at[j]],
                                                ss_cur, add=True)
                scats[IB - 2].wait()
                scats[IB - 1].wait()
                return carry

            lax.fori_loop(0, RPT // IB, group, 0)
            plsc.subcore_barrier()
            pltpu.sync_copy(acc.at[pl.ds(sid * RN, RN)],
                            out_hbm.at[slot, pl.ds(sid * RN, RN)])

        @pl.when(cid == 0)
        def _():
            run_chunk(pair[0], 0)

        @pl.when(cid == 1)
        def _():
            run_chunk(pair[1], 1)

    return gs_kernel


def _dinv_block(p_ref):
    deg = p_ref[0, :] + p_ref[1, :] + 1.0
    return lax.rsqrt(deg)[:, None]


def _mm1(x, W1, degp):
    N, K = x.shape
    nc1 = W1.shape[1] // FC

    def body(x_ref, w_ref, p_ref, o_ref):
        dinv = _dinv_block(p_ref)
        o_ref[0] = jnp.dot(x_ref[...], w_ref[...],
                           preferred_element_type=jnp.float32) * dinv

    return pl.pallas_call(
        body,
        grid=(N // RB, nc1),
        in_specs=[
            pl.BlockSpec((RB, K), lambda i, c: (i, 0)),
            pl.BlockSpec((K, FC), lambda i, c: (0, c)),
            pl.BlockSpec((2, RB), lambda i, c: (0, i)),
        ],
        out_specs=pl.BlockSpec((1, RB, FC), lambda i, c: (c, i, 0)),
        out_shape=jax.ShapeDtypeStruct((nc1, N, FC), jnp.float32),
    )(x, W1, degp)


def _mm2_partial(Tpair, degp, b1, W2, pair, prev=None, final=False):
    _, N, _ = Tpair.shape
    F1 = W2.shape[0]
    zc = W2.shape[1] // FC

    def body(*refs):
        if prev is None:
            t_ref, p_ref, b_ref, w_ref, o_ref = refs
        else:
            t_ref, p_ref, b_ref, w_ref, pr_ref, o_ref, dv_ref = refs
        dinv = _dinv_block(p_ref)
        acc = jnp.zeros((RB, FC), jnp.float32) if prev is None else pr_ref[0]
        for j, cj in enumerate(pair):
            h = jnp.maximum(t_ref[j] * dinv + b_ref[0, cj * FC:(cj + 1) * FC][None, :], 0.0)
            acc = acc + jnp.dot(h, w_ref[cj * FC:(cj + 1) * FC, :],
                                preferred_element_type=jnp.float32)
        if final:
            o_ref[0] = acc * dinv
            dv_ref[...] = dinv
        else:
            o_ref[0] = acc

    in_specs = [
        pl.BlockSpec((2, RB, FC), lambda i, c: (0, i, 0)),
        pl.BlockSpec((2, RB), lambda i, c: (0, i)),
        pl.BlockSpec((1, F1), lambda i, c: (0, 0)),
        pl.BlockSpec((F1, FC), lambda i, c: (0, c)),
    ]
    args = [Tpair, degp, b1.reshape(1, F1), W2]
    if prev is not None:
        in_specs.append(pl.BlockSpec((1, RB, FC), lambda i, c: (c, i, 0)))
        args.append(prev)
    out_specs = pl.BlockSpec((1, RB, FC), lambda i, c: (c, i, 0))
    out_shape = jax.ShapeDtypeStruct((zc, N, FC), jnp.float32)
    if final:
        out_specs = (out_specs, pl.BlockSpec((RB, 1), lambda i, c: (i, 0)))
        out_shape = (out_shape, jax.ShapeDtypeStruct((N, 1), jnp.float32))
    return pl.pallas_call(
        body,
        grid=(N // RB, zc),
        in_specs=in_specs,
        out_specs=out_specs,
        out_shape=out_shape,
    )(*args)


def _combine(T2, dinv, b2, N):
    zc = T2.shape[0]
    CB = 1000

    def body(t_ref, d_ref, b_ref, o_ref):
        o_ref[...] = t_ref[0] * d_ref[...] + b_ref[0][None, :]

    return pl.pallas_call(
        body,
        grid=(N // CB, zc),
        in_specs=[
            pl.BlockSpec((1, CB, FC), lambda i, c: (c, i, 0)),
            pl.BlockSpec((CB, 1), lambda i, c: (i, 0)),
            pl.BlockSpec((1, FC), lambda i, c: (0, c)),
        ],
        out_specs=pl.BlockSpec((CB, FC), lambda i, c: (i, c)),
        out_shape=jax.ShapeDtypeStruct((N, zc * FC), jnp.float32),
    )(T2, dinv, b2.reshape(1, -1))


def kernel(x, edge_index, W1, b1, W2, b2):
    N = x.shape[0]
    E = edge_index.shape[1]
    xp = jnp.pad(x, ((0, NPAD - N), (0, 0)))
    edge2d = edge_index.reshape(2 * (E // EB), EB)
    ones = jnp.ones((EB,), jnp.float32)
    zeros = jnp.zeros((NPAD,), jnp.float32)

    degp = _make_deg(E)(edge2d, ones, zeros)
    G1 = _mm1(xp, W1, degp)
    T1a = _make_gather_scatter(E, 4, (0, 2))(G1, edge2d)
    T1b = _make_gather_scatter(E, 4, (1, 3))(G1, edge2d)
    Pa = _mm2_partial(T1a, degp, b1, W2, (0, 2))
    G2, dinv = _mm2_partial(T1b, degp, b1, W2, (1, 3), prev=Pa, final=True)
    T2 = _make_gather_scatter(E, 2, (0, 1))(G2, edge2d)
    return _combine(T2, dinv, b2, N)
